# baseline (speedup 1.0000x reference)
import functools

import jax
import jax.numpy as jnp
from jax import lax
from jax.experimental import pallas as pl
from jax.experimental.pallas import tpu as pltpu
from jax.experimental.pallas import tpu_sc as plsc

_LANES = 16
_CHUNK_ROWS = 128
_NBUF = 2


def _sc_diffusion(x_start, noise, table_a, table_b, timesteps):
    num_steps = table_a.shape[0]
    batch, dim = x_start.shape
    groups = dim // _LANES
    mesh = plsc.VectorSubcoreMesh(core_axis_name="c", subcore_axis_name="s")
    num_workers = mesh.num_cores * mesh.num_subcores
    bpw = batch // num_workers
    num_chunks = bpw // _CHUNK_ROWS

    @functools.partial(
        pl.kernel,
        out_type=jax.ShapeDtypeStruct((batch, dim), jnp.float32),
        mesh=mesh,
        compiler_params=pltpu.CompilerParams(needs_layout_passes=False),
        scratch_types=[
            pltpu.VMEM((bpw,), jnp.int32),
            pltpu.VMEM((num_steps,), jnp.float32),
            pltpu.VMEM((num_steps,), jnp.float32),
            pltpu.VMEM((bpw,), jnp.float32),
            pltpu.VMEM((bpw,), jnp.float32),
            pltpu.VMEM((_NBUF, _CHUNK_ROWS, dim), jnp.float32),
            pltpu.VMEM((_NBUF, _CHUNK_ROWS, dim), jnp.float32),
            pltpu.VMEM((_NBUF, _CHUNK_ROWS, dim), jnp.float32),
            pltpu.SemaphoreType.DMA((_NBUF,)),
            pltpu.SemaphoreType.DMA((_NBUF,)),
        ],
    )
    def body(x_hbm, n_hbm, ta_hbm, tb_hbm, ts_hbm, o_hbm,
             idx_v, ta_v, tb_v, av_v, bv_v, xb, nb, ob, lsem, ssem):
        wid = lax.axis_index("s") * mesh.num_cores + lax.axis_index("c")
        base = wid * bpw

        def in_copies(c, b):
            r0 = base + c * _CHUNK_ROWS
            return (
                pltpu.make_async_copy(
                    x_hbm.at[pl.ds(r0, _CHUNK_ROWS), :], xb.at[b], lsem.at[b]),
                pltpu.make_async_copy(
                    n_hbm.at[pl.ds(r0, _CHUNK_ROWS), :], nb.at[b], lsem.at[b]),
            )

        def out_copy(c, b):
            r0 = base + c * _CHUNK_ROWS
            return pltpu.make_async_copy(
                ob.at[b], o_hbm.at[pl.ds(r0, _CHUNK_ROWS), :], ssem.at[b])

        for b in range(_NBUF):
            for cp in in_copies(b, b):
                cp.start()

        pltpu.sync_copy(ts_hbm.at[pl.ds(base, bpw)], idx_v)
        pltpu.sync_copy(ta_hbm, ta_v)
        pltpu.sync_copy(tb_hbm, tb_v)

        @plsc.parallel_loop(0, bpw, _LANES, unroll=4)
        def gather_body(off):
            iv = idx_v[pl.ds(off, _LANES)]
            av_v[pl.ds(off, _LANES)] = plsc.load_gather(ta_v, [iv])
            bv_v[pl.ds(off, _LANES)] = plsc.load_gather(tb_v, [iv])

        for c in range(num_chunks):
            b = c % _NBUF
            for cp in in_copies(c, b):
                cp.wait()
            if c >= _NBUF:
                out_copy(c - _NBUF, b).wait()

            @plsc.parallel_loop(0, _CHUNK_ROWS, unroll=8)
            def row_body(r):
                row = c * _CHUNK_ROWS + r
                ridx = jnp.full((_LANES,), row, jnp.int32)
                av = plsc.load_gather(av_v, [ridx])
                bv = plsc.load_gather(bv_v, [ridx])
                for j in range(groups):
                    sl = pl.ds(j * _LANES, _LANES)
                    ob[b, r, sl] = av * xb[b, r, sl] + bv * nb[b, r, sl]

            out_copy(c, b).start()
            if c + _NBUF < num_chunks:
                for cp in in_copies(c + _NBUF, b):
                    cp.start()

        for c in range(num_chunks - _NBUF, num_chunks):
            out_copy(c, c % _NBUF).wait()

    return body(x_start, noise, table_a, table_b, timesteps)


def kernel(x_start, noise, sqrt_alphas_cumprod, sqrt_one_minus_alphas_cumprod,
           timesteps):
    return _sc_diffusion(x_start, noise, sqrt_alphas_cumprod,
                         sqrt_one_minus_alphas_cumprod,
                         timesteps.astype(jnp.int32))
